# baseline (speedup 1.0000x reference)
import functools

import jax
import jax.numpy as jnp
from jax import lax
from jax.experimental import pallas as pl
from jax.experimental.pallas import tpu as pltpu
from jax.experimental.pallas import tpu_sc as plsc

EMB_N = 8192
EMB_D = 256
BM = 512
NCHUNK = 4
CW = EMB_N // NCHUNK
NSLICE = 12


def _e2_body(e_ref, e2_ref):
    e = e_ref[...]
    e2_ref[...] = lax.dot_general(
        jnp.ones((8, EMB_D), jnp.float32), e * e,
        (((1,), (1,)), ((), ())),
        precision=lax.Precision.HIGHEST,
        preferred_element_type=jnp.float32)


def _e2_kernel(e):
    return pl.pallas_call(
        _e2_body,
        out_shape=jax.ShapeDtypeStruct((8, EMB_N), jnp.float32),
    )(e)


def _vq_dist_argmin_body(x_ref, e_ref, e2_ref, idx_ref, loss_ref, acc_ref):
    m = pl.program_id(0)
    x = x_ref[...]

    @pl.when(m == 0)
    def _init():
        acc_ref[0] = 0.0

    x2 = jnp.sum(x * x, axis=1, keepdims=True)
    xm2 = -2.0 * x
    iota = lax.broadcasted_iota(jnp.int32, (BM, CW), 1).astype(jnp.float32)
    best_v = None
    best_i = None
    for j in range(NCHUNK):
        ej = e_ref[pl.ds(j * CW, CW), :]
        t = lax.dot_general(xm2, ej, (((1,), (1,)), ((), ())),
                            preferred_element_type=jnp.float32)
        d = (x2 + e2_ref[0:1, pl.ds(j * CW, CW)]) + t
        mv = jnp.min(d, axis=1, keepdims=True)
        iv = jnp.min(jnp.where(d == mv, iota, jnp.float32(EMB_N)),
                     axis=1, keepdims=True) + jnp.float32(j * CW)
        if j == 0:
            best_v, best_i = mv, iv
        else:
            upd = mv < best_v
            best_i = jnp.where(upd, iv, best_i)
            best_v = jnp.where(upd, mv, best_v)
    idx_ref[...] = best_i[:, 0].astype(jnp.int32)
    acc_ref[0] += jnp.sum(best_v)

    @pl.when(m == pl.num_programs(0) - 1)
    def _fin():
        loss_ref[0] = acc_ref[0]


def _dist_argmin(x, e, e2, num_rows):
    grid = (num_rows // BM,)
    idx, loss_sum = pl.pallas_call(
        _vq_dist_argmin_body,
        grid=grid,
        in_specs=[
            pl.BlockSpec((BM, EMB_D), lambda m: (m, 0)),
            pl.BlockSpec((EMB_N, EMB_D), lambda m: (0, 0)),
            pl.BlockSpec((8, EMB_N), lambda m: (0, 0)),
        ],
        out_specs=[
            pl.BlockSpec((BM,), lambda m: (m,)),
            pl.BlockSpec(memory_space=pltpu.SMEM),
        ],
        out_shape=[
            jax.ShapeDtypeStruct((num_rows,), jnp.int32),
            jax.ShapeDtypeStruct((1,), jnp.float32),
        ],
        scratch_shapes=[pltpu.SMEM((1,), jnp.float32)],
    )(x, e, e2)
    return idx, loss_sum


def _sc_gather(table, idx, num_rows):
    info = plsc.get_sparse_core_info()
    nc, ns = info.num_cores, info.num_subcores
    nw = nc * ns
    rows_per_w = num_rows // nw
    chunk = min(rows_per_w, 288)
    n_chunks = rows_per_w // chunk
    mesh = plsc.VectorSubcoreMesh(core_axis_name="c", subcore_axis_name="s")

    @functools.partial(
        pl.kernel, mesh=mesh,
        out_type=jax.ShapeDtypeStruct((num_rows, EMB_D), jnp.float32),
        scratch_types=[
            pltpu.VMEM((chunk,), jnp.int32),
            pltpu.VMEM((chunk, EMB_D), jnp.float32),
            pltpu.SemaphoreType.DMA,
        ],
    )
    def gather_kernel(idx_hbm, table_hbm, out_hbm, idx_v, rows_v, sem):
        wid = lax.axis_index("s") * nc + lax.axis_index("c")
        for t in range(n_chunks):
            base = wid * rows_per_w + t * chunk
            pltpu.sync_copy(idx_hbm.at[pl.ds(base, chunk)], idx_v)
            pltpu.async_copy(table_hbm.at[idx_v], rows_v, sem).wait()
            pltpu.sync_copy(rows_v, out_hbm.at[pl.ds(base, chunk)])

    return gather_kernel(idx, table)


def kernel(inputs, embedding):
    x = inputs.reshape(-1, EMB_D)
    num_rows = x.shape[0]
    rows_s = num_rows // NSLICE
    e2 = _e2_kernel(embedding)
    idx_parts, q_parts, loss_parts = [], [], []
    for s in range(NSLICE):
        xs = lax.slice_in_dim(x, s * rows_s, (s + 1) * rows_s, axis=0)
        idx_s, loss_s = _dist_argmin(xs, embedding, e2, rows_s)
        q_s = _sc_gather(embedding, idx_s, rows_s)
        idx_parts.append(idx_s)
        q_parts.append(q_s)
        loss_parts.append(loss_s[0])
    idx = jnp.concatenate(idx_parts)
    quantized = jnp.concatenate(q_parts)
    loss_sum = loss_parts[0]
    for p in loss_parts[1:]:
        loss_sum = loss_sum + p
    loss = (0.5 / (num_rows * EMB_D)) * loss_sum
    return (quantized.reshape(inputs.shape), idx[:, None], loss)


# speedup vs baseline: 1.0805x; 1.0805x over previous
import functools

import jax
import jax.numpy as jnp
from jax import lax
from jax.experimental import pallas as pl
from jax.experimental.pallas import tpu as pltpu
from jax.experimental.pallas import tpu_sc as plsc

EMB_N = 8192
EMB_D = 256
BM = 512
NCHUNK = 4
CW = EMB_N // NCHUNK
NSLICE = 6


def _e2_body(e_ref, e2_ref):
    e = e_ref[...]
    e2_ref[...] = lax.dot_general(
        jnp.ones((8, EMB_D), jnp.float32), e * e,
        (((1,), (1,)), ((), ())),
        precision=lax.Precision.HIGHEST,
        preferred_element_type=jnp.float32)


def _e2_kernel(e):
    return pl.pallas_call(
        _e2_body,
        out_shape=jax.ShapeDtypeStruct((8, EMB_N), jnp.float32),
    )(e)


def _vq_dist_argmin_body(x_ref, e_ref, e2_ref, idx_ref, loss_ref, acc_ref):
    m = pl.program_id(0)
    x = x_ref[...]

    @pl.when(m == 0)
    def _init():
        acc_ref[0] = 0.0

    x2 = jnp.sum(x * x, axis=1, keepdims=True)
    xm2 = -2.0 * x
    iota = lax.broadcasted_iota(jnp.int32, (BM, CW), 1).astype(jnp.float32)
    best_v = None
    best_i = None
    for j in range(NCHUNK):
        ej = e_ref[pl.ds(j * CW, CW), :]
        t = lax.dot_general(xm2, ej, (((1,), (1,)), ((), ())),
                            preferred_element_type=jnp.float32)
        d = (x2 + e2_ref[0:1, pl.ds(j * CW, CW)]) + t
        mv = jnp.min(d, axis=1, keepdims=True)
        iv = jnp.min(jnp.where(d == mv, iota, jnp.float32(EMB_N)),
                     axis=1, keepdims=True) + jnp.float32(j * CW)
        if j == 0:
            best_v, best_i = mv, iv
        else:
            upd = mv < best_v
            best_i = jnp.where(upd, iv, best_i)
            best_v = jnp.where(upd, mv, best_v)
    idx_ref[...] = best_i[:, 0].astype(jnp.int32)
    acc_ref[0] += jnp.sum(best_v)

    @pl.when(m == pl.num_programs(0) - 1)
    def _fin():
        loss_ref[0] = acc_ref[0]


def _dist_argmin(x, e, e2, num_rows):
    grid = (num_rows // BM,)
    idx, loss_sum = pl.pallas_call(
        _vq_dist_argmin_body,
        grid=grid,
        in_specs=[
            pl.BlockSpec((BM, EMB_D), lambda m: (m, 0)),
            pl.BlockSpec((EMB_N, EMB_D), lambda m: (0, 0)),
            pl.BlockSpec((8, EMB_N), lambda m: (0, 0)),
        ],
        out_specs=[
            pl.BlockSpec((BM,), lambda m: (m,)),
            pl.BlockSpec(memory_space=pltpu.SMEM),
        ],
        out_shape=[
            jax.ShapeDtypeStruct((num_rows,), jnp.int32),
            jax.ShapeDtypeStruct((1,), jnp.float32),
        ],
        scratch_shapes=[pltpu.SMEM((1,), jnp.float32)],
    )(x, e, e2)
    return idx, loss_sum


def _sc_gather(table, idx, num_rows):
    info = plsc.get_sparse_core_info()
    nc, ns = info.num_cores, info.num_subcores
    nw = nc * ns
    rows_per_w = num_rows // nw
    chunk = min(rows_per_w, 288)
    n_chunks = rows_per_w // chunk
    mesh = plsc.VectorSubcoreMesh(core_axis_name="c", subcore_axis_name="s")

    @functools.partial(
        pl.kernel, mesh=mesh,
        out_type=jax.ShapeDtypeStruct((num_rows, EMB_D), jnp.float32),
        scratch_types=[
            pltpu.VMEM((chunk,), jnp.int32),
            pltpu.VMEM((chunk, EMB_D), jnp.float32),
            pltpu.SemaphoreType.DMA,
        ],
    )
    def gather_kernel(idx_hbm, table_hbm, out_hbm, idx_v, rows_v, sem):
        wid = lax.axis_index("s") * nc + lax.axis_index("c")
        for t in range(n_chunks):
            base = wid * rows_per_w + t * chunk
            pltpu.sync_copy(idx_hbm.at[pl.ds(base, chunk)], idx_v)
            pltpu.async_copy(table_hbm.at[idx_v], rows_v, sem).wait()
            pltpu.sync_copy(rows_v, out_hbm.at[pl.ds(base, chunk)])

    return gather_kernel(idx, table)


def kernel(inputs, embedding):
    x = inputs.reshape(-1, EMB_D)
    num_rows = x.shape[0]
    rows_s = num_rows // NSLICE
    e2 = _e2_kernel(embedding)
    idx_parts, q_parts, loss_parts = [], [], []
    for s in range(NSLICE):
        xs = lax.slice_in_dim(x, s * rows_s, (s + 1) * rows_s, axis=0)
        idx_s, loss_s = _dist_argmin(xs, embedding, e2, rows_s)
        q_s = _sc_gather(embedding, idx_s, rows_s)
        idx_parts.append(idx_s)
        q_parts.append(q_s)
        loss_parts.append(loss_s[0])
    idx = jnp.concatenate(idx_parts)
    quantized = jnp.concatenate(q_parts)
    loss_sum = loss_parts[0]
    for p in loss_parts[1:]:
        loss_sum = loss_sum + p
    loss = (0.5 / (num_rows * EMB_D)) * loss_sum
    return (quantized.reshape(inputs.shape), idx[:, None], loss)
